# Initial kernel scaffold; baseline (speedup 1.0000x reference)
import jax
import jax.numpy as jnp
from jax.experimental import pallas as pl


def kernel(xh_atoms, xh_residues, t, mask_atoms, mask_residues, params):
    raise NotImplementedError("write your pallas kernel here")


# speedup vs baseline: 9.1294x; 9.1294x over previous
import functools

import jax
import jax.numpy as jnp
from jax.experimental import pallas as pl
from jax.experimental.pallas import tpu as pltpu

NA = 2048
NR = 2048
NDIM = 3
ATOM_NF = 16
RES_NF = 21
JOINT = 16
HID = 64
NL = 4
NBATCH = 32
NORM = 100.0
N = NA + NR
TILE = 128
NT = N // TILE


def _silu(x):
    return x * jax.nn.sigmoid(x)


def _egnn_kernel(ranges_ref, xa_ref, xr_ref, mrow_ref, mcol_ref,
                 aew1, aeb1, aew2, aeb2, rew1, reb1, rew2, reb2,
                 embw, emb_bias, lw, outw, outb,
                 adw1, adb1, adw2, adb2, adw3, adb3,
                 out_ref, hh_scr, a_scr, b_scr, agg_scr):
    ha = _silu(xa_ref[:, NDIM:] @ aew1[:] + aeb1[:]) @ aew2[:] + aeb2[:]
    hr = _silu(xr_ref[:, NDIM:] @ rew1[:] + reb1[:]) @ rew2[:] + reb2[:]
    hh_scr[0:NA, :] = (xa_ref[:, :NDIM] @ embw[0:NDIM, :]
                       + ha @ embw[NDIM:NDIM + JOINT, :] + emb_bias[:])
    hh_scr[NA:N, :] = (xr_ref[:, :NDIM] @ embw[0:NDIM, :]
                       + hr @ embw[NDIM:NDIM + JOINT, :] + emb_bias[:])

    for l in range(NL):
        w1, b1, w2, b2, nw1, nb1, nw2, nb2 = [r[:] for r in lw[l]]
        hh = hh_scr[:]
        a_scr[:] = hh @ w1[:HID, :] + b1
        b_scr[:] = hh @ w1[HID:, :]

        def row_body(r, _, w2=w2, b2=b2):
            a_tile = a_scr[pl.ds(r * TILE, TILE), :]
            mrow = mrow_ref[pl.ds(r * TILE, TILE), :]

            def j_body(j, acc):
                b_tile = b_scr[pl.ds(j * TILE, TILE), :]
                mcol = mrow_ref[pl.ds(j * TILE, TILE), :]
                e = _silu(a_tile[:, None, :] + b_tile[None, :, :])
                m = _silu(jax.lax.dot_general(
                    e, w2, (((2,), (0,)), ((), ())),
                    preferred_element_type=jnp.float32) + b2)
                adj = mrow[:, None, :] == mcol[None, :, :]
                return acc + jnp.sum(jnp.where(adj, m, 0.0), axis=1)

            acc = jnp.zeros((TILE, HID), jnp.float32)
            acc = jax.lax.fori_loop(ranges_ref[r, 0], ranges_ref[r, 1],
                                    j_body, acc)
            acc = jax.lax.fori_loop(ranges_ref[r, 2], ranges_ref[r, 3],
                                    j_body, acc)
            agg_scr[pl.ds(r * TILE, TILE), :] = acc
            return 0

        jax.lax.fori_loop(0, NT, row_body, 0)

        hh = hh_scr[:]
        upd = _silu(hh @ nw1[:HID, :] + (agg_scr[:] * (1.0 / NORM)) @ nw1[HID:, :]
                    + nb1) @ nw2 + nb2
        hh_scr[:] = hh + upd

    out = hh_scr[:] @ outw[:] + outb[:]
    vel = out[:, :NDIM]
    hf = out[:, NDIM:NDIM + JOINT]
    seg_ids = jax.lax.broadcasted_iota(jnp.int32, (N, NBATCH), 1)
    oh = (mrow_ref[:] == seg_ids).astype(jnp.float32)
    seg = jax.lax.dot_general(oh, vel, (((0,), (0,)), ((), ())),
                              preferred_element_type=jnp.float32)
    cnt = jnp.sum(oh, axis=0, keepdims=True)
    mean = seg / jnp.maximum(cnt.T, 1.0)
    velc = vel - oh @ mean
    hfa = hf[0:NA, :]
    d = _silu(hfa @ adw1[:] + adb1[:])
    d = _silu(d @ adw2[:] + adb2[:])
    d = d @ adw3[:] + adb3[:]
    out_ref[:] = velc[0:NA, :] + d


def kernel(xh_atoms, xh_residues, t, mask_atoms, mask_residues, params):
    p = params
    ma = mask_atoms.astype(jnp.int32)
    mr = mask_residues.astype(jnp.int32)
    m_full = jnp.concatenate([ma, mr])
    mrow = m_full[:, None]
    mcol = m_full[None, :]

    mt = m_full.reshape(NT, TILE)
    lo = mt[:, 0]
    hi = mt[:, -1]
    a_s = jnp.searchsorted(ma, lo, side='left').astype(jnp.int32)
    a_e = jnp.searchsorted(ma, hi, side='right').astype(jnp.int32)
    r_s = jnp.searchsorted(mr, lo, side='left').astype(jnp.int32)
    r_e = jnp.searchsorted(mr, hi, side='right').astype(jnp.int32)
    a_js = a_s // TILE
    a_je = -((-a_e) // TILE)
    r_js = NA // TILE + r_s // TILE
    r_je = NA // TILE + (-((-r_e) // TILE))
    ranges = jnp.stack([a_js, a_je, r_js, r_je], axis=1)

    emb_bias = (p['embb'] + t[0] * p['embw'][NDIM + JOINT])[None, :]

    lw_names = []
    for l in range(NL):
        lw_names.append(['ew1_%d' % l, 'eb1_%d' % l, 'ew2_%d' % l, 'eb2_%d' % l,
                         'nw1_%d' % l, 'nb1_%d' % l, 'nw2_%d' % l, 'nb2_%d' % l])
    lw_vals = [[p[n] for n in names] for names in lw_names]

    flat_params = ([p['aew1'], p['aeb1'], p['aew2'], p['aeb2'],
                    p['rew1'], p['reb1'], p['rew2'], p['reb2'],
                    p['embw'], emb_bias]
                   + [w for layer in lw_vals for w in layer]
                   + [p['outw'], p['outb'],
                      p['adw1'], p['adb1'], p['adw2'], p['adb2'],
                      p['adw3'], p['adb3']])

    def kern_wrap(ranges_ref, xa_ref, xr_ref, mrow_ref, mcol_ref, *rest):
        prm = list(rest[:len(flat_params)])
        out_ref = rest[len(flat_params)]
        scr = rest[len(flat_params) + 1:]
        aew1, aeb1, aew2, aeb2, rew1, reb1, rew2, reb2, embw, emb_b = prm[:10]
        lw = [prm[10 + 8 * l:10 + 8 * (l + 1)] for l in range(NL)]
        outw, outb, adw1, adb1, adw2, adb2, adw3, adb3 = prm[10 + 8 * NL:]
        _egnn_kernel(ranges_ref, xa_ref, xr_ref, mrow_ref, mcol_ref,
                     aew1, aeb1, aew2, aeb2, rew1, reb1, rew2, reb2,
                     embw, emb_b, lw, outw, outb,
                     adw1, adb1, adw2, adb2, adw3, adb3,
                     out_ref, *scr)

    n_in = 5 + len(flat_params)
    in_specs = ([pl.BlockSpec(memory_space=pltpu.SMEM)]
                + [pl.BlockSpec(memory_space=pltpu.VMEM)] * (n_in - 1))
    out = pl.pallas_call(
        kern_wrap,
        out_shape=jax.ShapeDtypeStruct((NA, NDIM), jnp.float32),
        in_specs=in_specs,
        out_specs=pl.BlockSpec(memory_space=pltpu.VMEM),
        scratch_shapes=[
            pltpu.VMEM((N, HID), jnp.float32),
            pltpu.VMEM((N, HID), jnp.float32),
            pltpu.VMEM((N, HID), jnp.float32),
            pltpu.VMEM((N, HID), jnp.float32),
        ],
    )(ranges, xh_atoms, xh_residues, mrow, mcol, *flat_params)
    return out
